# baseline (speedup 1.0000x reference)
import functools

import jax
import jax.numpy as jnp
from jax import lax
from jax.experimental import pallas as pl
from jax.experimental.pallas import tpu as pltpu
from jax.experimental.pallas import tpu_sc as plsc

_NC = 2
_NS = 16
_NW = _NC * _NS
_CH = 40


@functools.lru_cache(maxsize=None)
def _sc_agg(n_nodes: int, n_edges: int, feat: int, n_pad: int, with_deg: bool):
    epw = n_edges // _NW
    rpt = n_pad // _NS
    nch = epw // _CH
    assert epw * _NW == n_edges and nch * _CH == epw
    assert rpt * _NS == n_pad and rpt % 8 == 0

    mesh = plsc.VectorSubcoreMesh(core_axis_name="c", subcore_axis_name="s")
    out_type = [jax.ShapeDtypeStruct((_NC, n_pad, feat), jnp.float32)]
    if with_deg:
        out_type.append(jax.ShapeDtypeStruct((_NC * n_pad,), jnp.float32))

    nbuf = 3
    nmain = (nch // nbuf) * nbuf
    nhalf = nch // 2
    ehalf = epw // 2
    scratch_types = [
        pltpu.VMEM((ehalf,), jnp.int32),
        pltpu.VMEM((nhalf, _CH), jnp.int32),
    ] + [pltpu.VMEM((_CH, feat), jnp.float32)] * nbuf + [
        pltpu.VMEM((_CH,), jnp.float32),
    ] + [pltpu.SemaphoreType.DMA] * (3 * nbuf) + [
        pltpu.VMEM_SHARED((n_pad, feat), jnp.float32),
        pltpu.VMEM_SHARED((n_pad,), jnp.float32),
    ]

    @functools.partial(pl.kernel, mesh=mesh, out_type=out_type,
                       scratch_types=scratch_types)
    def agg(x_hbm, src_hbm, dst3_hbm, *refs):
        if with_deg:
            out_hbm, deg_hbm = refs[0], refs[1]
            refs = refs[2:]
        else:
            out_hbm, deg_hbm = refs[0], None
            refs = refs[1:]
        srcbuf, dstbuf = refs[:2]
        rows = refs[2:2 + nbuf]
        ones = refs[2 + nbuf]
        base = 3 + nbuf
        semg = refs[base:base + nbuf]
        sems = refs[base + nbuf:base + 2 * nbuf]
        semd = refs[base + 2 * nbuf:base + 3 * nbuf]
        acc, dacc = refs[base + 3 * nbuf:]

        c = lax.axis_index("c")
        s = lax.axis_index("s")
        wid = c * _NS + s

        @pl.loop(0, _CH)
        def _zrow(r):
            for c5 in range(feat // 16):
                rows[0][r, pl.ds(c5 * 16, 16)] = jnp.zeros((16,), jnp.float32)

        @pl.loop(0, rpt, step=_CH)
        def _zacc(j):
            pltpu.sync_copy(rows[0], acc.at[pl.ds(s * rpt + j, _CH)])

        if with_deg:
            @pl.loop(0, rpt, step=feat)
            def _zdeg(j):
                pltpu.sync_copy(rows[0].at[0],
                                dacc.at[pl.ds(s * rpt + j, feat)])

            @pl.loop(0, _CH, step=16)
            def _fill(i):
                ones[pl.ds(i, 16)] = jnp.ones((16,), jnp.float32)

        plsc.subcore_barrier()

        def gather(ci, k):
            pltpu.async_copy(x_hbm.at[srcbuf.at[pl.ds(ci * _CH, _CH)]],
                             rows[k], semg[k])

        def wait_gather(k):
            pltpu.make_async_copy(x_hbm.at[pl.ds(0, _CH)], rows[k],
                                  semg[k]).wait()

        nmainh = (nhalf // nbuf) * nbuf
        for half in range(2):
            hs = pltpu.async_copy(
                src_hbm.at[pl.ds(wid * epw + half * ehalf, ehalf)], srcbuf,
                semg[1])
            hd0 = pltpu.async_copy(dst3_hbm.at[wid, half], dstbuf, sems[0])
            hs.wait()
            hd0.wait()

            for k in range(min(nbuf, nhalf)):
                gather(k, k)

            @pl.loop(0, nmainh, step=nbuf)
            def _group(c0):
                handles = []
                for k in range(nbuf):
                    wait_gather(k)
                    h = pltpu.async_copy(rows[k], acc.at[dstbuf.at[c0 + k]],
                                         sems[k], add=True)
                    hd = None
                    if with_deg:
                        hd = pltpu.async_copy(ones,
                                              dacc.at[dstbuf.at[c0 + k]],
                                              semd[k], add=True)
                    handles.append((h, hd))
                for k in range(nbuf):
                    h, hd = handles[k]
                    h.wait()
                    if hd is not None:
                        hd.wait()

                    @pl.when(c0 + nbuf + k < nhalf)
                    def _(k=k):
                        gather(c0 + nbuf + k, k)

            for ci in range(nmainh, nhalf):
                k = ci - nmainh
                wait_gather(k)
                pltpu.sync_copy(rows[k], acc.at[dstbuf.at[ci]], add=True)
                if with_deg:
                    pltpu.sync_copy(ones, dacc.at[dstbuf.at[ci]], add=True)

        plsc.subcore_barrier()

        ho = pltpu.async_copy(acc.at[pl.ds(s * rpt, rpt)],
                              out_hbm.at[c, pl.ds(s * rpt, rpt)], semg[0])
        if with_deg:
            pltpu.sync_copy(dacc.at[pl.ds(s * rpt, rpt)],
                            deg_hbm.at[pl.ds(c * n_pad + s * rpt, rpt)])
        ho.wait()

    return agg


@functools.lru_cache(maxsize=None)
def _tc_layer(n_nodes: int, n_pad: int, feat: int, out_feat: int, act: str):
    rblk = 2000
    assert n_nodes % rblk == 0

    def body(p_ref, dg_ref, x_ref, wl_ref, b_ref, wr_ref, o_ref):
        agg = p_ref[0] + p_ref[1]
        deg = dg_ref[0] + dg_ref[1]
        dinv = 1.0 / jnp.maximum(deg, 1.0)
        mean = agg * dinv
        y = jnp.dot(mean, wl_ref[...], preferred_element_type=jnp.float32)
        y = y + jnp.dot(x_ref[...], wr_ref[...],
                        preferred_element_type=jnp.float32)
        y = y + b_ref[...]
        if act == "relu":
            o_ref[...] = jnp.maximum(y, 0.0)
        else:
            m = jnp.max(y, axis=1, keepdims=True)
            lse = jnp.log(jnp.sum(jnp.exp(y - m), axis=1, keepdims=True)) + m
            o_ref[...] = y - lse

    return pl.pallas_call(
        body,
        grid=(n_nodes // rblk,),
        in_specs=[
            pl.BlockSpec((_NC, rblk, feat), lambda i: (0, i, 0)),
            pl.BlockSpec((_NC, rblk, 1), lambda i: (0, i, 0)),
            pl.BlockSpec((rblk, feat), lambda i: (i, 0)),
            pl.BlockSpec((feat, out_feat), lambda i: (0, 0)),
            pl.BlockSpec((1, out_feat), lambda i: (0, 0)),
            pl.BlockSpec((feat, out_feat), lambda i: (0, 0)),
        ],
        out_specs=pl.BlockSpec((rblk, out_feat), lambda i: (i, 0)),
        out_shape=jax.ShapeDtypeStruct((n_nodes, out_feat), jnp.float32),
    )


def kernel(x, edge_index, W1_l, b1, W1_r, W2_l, b2, W2_r):
    n, d = x.shape
    e = edge_index.shape[1]
    h = W1_l.shape[1]
    o = W2_l.shape[1]
    n_pad = 10240

    src = edge_index[0]
    dst = edge_index[1].reshape(_NW, 2, (e // _NW) // _CH // 2, _CH)
    p1, dg = _sc_agg(n, e, d, n_pad, True)(x, src, dst)
    dg3 = dg.reshape(_NC, n_pad, 1)
    hid = _tc_layer(n, n_pad, d, h, "relu")(
        p1, dg3, x, W1_l, b1.reshape(1, h), W1_r)
    (p2,) = _sc_agg(n, e, h, n_pad, False)(hid, src, dst)
    out = _tc_layer(n, n_pad, h, o, "ls")(
        p2, dg3, hid, W2_l, b2.reshape(1, o), W2_r)
    return out


# speedup vs baseline: 1.1433x; 1.1433x over previous
import functools

import jax
import jax.numpy as jnp
from jax import lax
from jax.experimental import pallas as pl
from jax.experimental.pallas import tpu as pltpu
from jax.experimental.pallas import tpu_sc as plsc

_NC = 2
_NS = 16
_NW = _NC * _NS
_CH = 80


@functools.lru_cache(maxsize=None)
def _sc_agg(n_nodes: int, n_edges: int, feat: int, n_pad: int, with_deg: bool):
    epw = n_edges // _NW
    rpt = n_pad // _NS
    nch = epw // _CH
    assert epw * _NW == n_edges and nch * _CH == epw
    assert rpt * _NS == n_pad and rpt % 8 == 0

    mesh = plsc.VectorSubcoreMesh(core_axis_name="c", subcore_axis_name="s")
    out_type = [jax.ShapeDtypeStruct((_NC, n_pad, feat), jnp.float32)]
    if with_deg:
        out_type.append(jax.ShapeDtypeStruct((_NC * n_pad,), jnp.float32))

    nbuf = 2
    nmain = (nch // nbuf) * nbuf
    scratch_types = [
        pltpu.VMEM((epw,), jnp.int32),
        pltpu.VMEM((nch, _CH), jnp.int32),
    ] + [pltpu.VMEM((_CH, feat), jnp.float32)] * nbuf + [
        pltpu.VMEM((_CH,), jnp.float32),
    ] + [pltpu.SemaphoreType.DMA] * (3 * nbuf) + [
        pltpu.VMEM_SHARED((n_pad, feat), jnp.float32),
        pltpu.VMEM_SHARED((n_pad,), jnp.float32),
    ]

    @functools.partial(pl.kernel, mesh=mesh, out_type=out_type,
                       scratch_types=scratch_types)
    def agg(x_hbm, src_hbm, dst3_hbm, *refs):
        if with_deg:
            out_hbm, deg_hbm = refs[0], refs[1]
            refs = refs[2:]
        else:
            out_hbm, deg_hbm = refs[0], None
            refs = refs[1:]
        srcbuf, dstbuf = refs[:2]
        rows = refs[2:2 + nbuf]
        ones = refs[2 + nbuf]
        base = 3 + nbuf
        semg = refs[base:base + nbuf]
        sems = refs[base + nbuf:base + 2 * nbuf]
        semd = refs[base + 2 * nbuf:base + 3 * nbuf]
        acc, dacc = refs[base + 3 * nbuf:]

        c = lax.axis_index("c")
        s = lax.axis_index("s")
        wid = c * _NS + s

        hs = pltpu.async_copy(src_hbm.at[pl.ds(wid * epw, epw)], srcbuf,
                              semg[1])
        hd0 = pltpu.async_copy(dst3_hbm.at[wid], dstbuf, sems[0])

        @pl.loop(0, _CH)
        def _zrow(r):
            for c5 in range(feat // 16):
                rows[0][r, pl.ds(c5 * 16, 16)] = jnp.zeros((16,), jnp.float32)

        @pl.loop(0, rpt, step=_CH)
        def _zacc(j):
            pltpu.sync_copy(rows[0], acc.at[pl.ds(s * rpt + j, _CH)])

        if with_deg:
            @pl.loop(0, rpt, step=feat)
            def _zdeg(j):
                pltpu.sync_copy(rows[0].at[0],
                                dacc.at[pl.ds(s * rpt + j, feat)])

            @pl.loop(0, _CH, step=16)
            def _fill(i):
                ones[pl.ds(i, 16)] = jnp.ones((16,), jnp.float32)

        hs.wait()
        hd0.wait()
        plsc.subcore_barrier()

        def gather(ci, k):
            pltpu.async_copy(x_hbm.at[srcbuf.at[pl.ds(ci * _CH, _CH)]],
                             rows[k], semg[k])

        def wait_gather(k):
            pltpu.make_async_copy(x_hbm.at[pl.ds(0, _CH)], rows[k],
                                  semg[k]).wait()

        for k in range(min(nbuf, nch)):
            gather(k, k)

        @pl.loop(0, nch, step=nbuf)
        def _group(c0):
            for k in range(nbuf):
                @pl.when(c0 + k < nch)
                def _(k=k):
                    wait_gather(k)
                    h = pltpu.async_copy(rows[k], acc.at[dstbuf.at[c0 + k]],
                                         sems[k], add=True)
                    hd = None
                    if with_deg:
                        hd = pltpu.async_copy(ones, dacc.at[dstbuf.at[c0 + k]],
                                              semd[k], add=True)
                    h.wait()
                    if hd is not None:
                        hd.wait()

                    @pl.when(c0 + nbuf + k < nch)
                    def _():
                        gather(c0 + nbuf + k, k)

        plsc.subcore_barrier()

        ho = pltpu.async_copy(acc.at[pl.ds(s * rpt, rpt)],
                              out_hbm.at[c, pl.ds(s * rpt, rpt)], semg[0])
        if with_deg:
            pltpu.sync_copy(dacc.at[pl.ds(s * rpt, rpt)],
                            deg_hbm.at[pl.ds(c * n_pad + s * rpt, rpt)])
        ho.wait()

    return agg


@functools.lru_cache(maxsize=None)
def _tc_layer(n_nodes: int, n_pad: int, feat: int, out_feat: int, act: str):
    rblk = 2000
    assert n_nodes % rblk == 0

    def body(p_ref, dg_ref, x_ref, wl_ref, b_ref, wr_ref, o_ref):
        agg = p_ref[0] + p_ref[1]
        deg = dg_ref[0] + dg_ref[1]
        dinv = 1.0 / jnp.maximum(deg, 1.0)
        mean = agg * dinv
        y = jnp.dot(mean, wl_ref[...], preferred_element_type=jnp.float32)
        y = y + jnp.dot(x_ref[...], wr_ref[...],
                        preferred_element_type=jnp.float32)
        y = y + b_ref[...]
        if act == "relu":
            o_ref[...] = jnp.maximum(y, 0.0)
        else:
            m = jnp.max(y, axis=1, keepdims=True)
            lse = jnp.log(jnp.sum(jnp.exp(y - m), axis=1, keepdims=True)) + m
            o_ref[...] = y - lse

    return pl.pallas_call(
        body,
        grid=(n_nodes // rblk,),
        in_specs=[
            pl.BlockSpec((_NC, rblk, feat), lambda i: (0, i, 0)),
            pl.BlockSpec((_NC, rblk, 1), lambda i: (0, i, 0)),
            pl.BlockSpec((rblk, feat), lambda i: (i, 0)),
            pl.BlockSpec((feat, out_feat), lambda i: (0, 0)),
            pl.BlockSpec((1, out_feat), lambda i: (0, 0)),
            pl.BlockSpec((feat, out_feat), lambda i: (0, 0)),
        ],
        out_specs=pl.BlockSpec((rblk, out_feat), lambda i: (i, 0)),
        out_shape=jax.ShapeDtypeStruct((n_nodes, out_feat), jnp.float32),
    )


def kernel(x, edge_index, W1_l, b1, W1_r, W2_l, b2, W2_r):
    n, d = x.shape
    e = edge_index.shape[1]
    h = W1_l.shape[1]
    o = W2_l.shape[1]
    n_pad = 10240

    src = edge_index[0]
    dst = edge_index[1].reshape(_NW, (e // _NW) // _CH, _CH)
    p1, dg = _sc_agg(n, e, d, n_pad, True)(x, src, dst)
    dg3 = dg.reshape(_NC, n_pad, 1)
    hid = _tc_layer(n, n_pad, d, h, "relu")(
        p1, dg3, x, W1_l, b1.reshape(1, h), W1_r)
    (p2,) = _sc_agg(n, e, h, n_pad, False)(hid, src, dst)
    out = _tc_layer(n, n_pad, h, o, "ls")(
        p2, dg3, hid, W2_l, b2.reshape(1, o), W2_r)
    return out
